# baseline (speedup 1.0000x reference)
import functools

import jax
import jax.numpy as jnp
from jax import lax
from jax.experimental import pallas as pl
from jax.experimental.pallas import tpu as pltpu
from jax.experimental.pallas import tpu_sc as plsc

_NUM_GRAPHS = 16
_EPS = 1e-5
_NC = 2
_NS = 16
_NW = _NC * _NS
_CHUNK = 128
_BLK_EDGES = 2048

_SC_PARAMS = pltpu.CompilerParams(needs_layout_passes=False)


def _mm_body(z_ref, w_ref, h_ref):
    h_ref[0] = jnp.maximum(
        lax.dot_general(z_ref[...], w_ref[0], (((1,), (0,)), ((), ())),
                        preferred_element_type=jnp.float32),
        0.0)


def _relu_mm(zin, W, bm):
    npad, d = zin.shape
    num_rel, _, hid = W.shape
    return pl.pallas_call(
        _mm_body,
        grid=(npad // bm, num_rel),
        in_specs=[
            pl.BlockSpec((bm, d), lambda i, r: (i, 0)),
            pl.BlockSpec((1, d, hid), lambda i, r: (r, 0, 0)),
        ],
        out_specs=pl.BlockSpec((1, bm, hid), lambda i, r: (r, i, 0)),
        out_shape=jax.ShapeDtypeStruct((num_rel, npad, hid), jnp.float32),
    )(zin, W)


def _iota16():
    return lax.iota(jnp.int32, 16)


def _make_filter_kernel(npad, ne_pad):
    rpt = npad // _NW
    nblocks_in = ne_pad // _BLK_EDGES
    cap_chunks = ne_pad // _CHUNK + 24
    cap_words = cap_chunks * _CHUNK
    trash_off = (cap_chunks - 18) * _CHUNK
    cnt_off = (cap_chunks - 1) * _CHUNK
    mesh = plsc.VectorSubcoreMesh(core_axis_name="c", subcore_axis_name="s")

    @functools.partial(
        pl.kernel, mesh=mesh,
        compiler_params=_SC_PARAMS,
        out_type=jax.ShapeDtypeStruct((_NW * cap_words,), jnp.int32),
        scratch_types=[
            pltpu.VMEM((16, _CHUNK), jnp.int32),
            pltpu.VMEM((16, _CHUNK), jnp.int32),
            pltpu.VMEM((4224,), jnp.int32),
        ],
    )
    def filter_kernel(gidx_hbm, src_hbm, list_hbm, gstage, sstage, stage):
        c = lax.axis_index("c")
        s = lax.axis_index("s")
        w = c * _NS + s
        lo = w * rpt
        wbase = w * cap_words
        iota = _iota16()
        dummy = jnp.full((16,), (npad - 1) * 512, jnp.int32)

        def block_body(b, carry):
            off, fwords = carry
            pltpu.sync_copy(gidx_hbm.at[pl.ds(b * 16, 16)], gstage)
            pltpu.sync_copy(src_hbm.at[pl.ds(b * 16, 16)], sstage)

            def vec_body(i, off_):
                r = i // 8
                cofs = (i % 8) * 16
                sv = sstage[r, pl.ds(cofs, 16)]
                gv = gstage[r, pl.ds(cofs, 16)]
                dloc = sv - lo
                mask = jnp.logical_and(dloc >= 0, dloc < rpt)
                rec = gv * 512 + dloc
                plsc.store_compressed(stage.at[pl.ds(off_, 16)], rec,
                                      mask=mask)
                return off_ + plsc.all_reduce_population_count(mask)[0]
            off = lax.fori_loop(0, 128, vec_body, off)

            nf = off // 2048
            dst = jnp.where(nf > 0, wbase + fwords, wbase + trash_off)
            pltpu.sync_copy(
                stage.at[pl.ds(0, 2048)],
                list_hbm.at[pl.ds(pl.multiple_of(dst, _CHUNK), 2048)])
            base = nf * 2048

            def move_body(i, carry_):
                stage[pl.ds(i * 16, 16)] = stage[pl.ds(base + i * 16, 16)]
                return carry_
            lax.fori_loop(0, 128, move_body, 0)
            return (off - nf * 2048, fwords + nf * 2048)

        off, fwords = lax.fori_loop(0, nblocks_in, block_body, (0, 0))

        padn = (1024 - off % 1024) % 1024

        def pad_body(t, off_):
            m = jnp.clip(padn - t * 16, 0, 16)
            plsc.store_compressed(stage.at[pl.ds(off_, 16)], dummy,
                                  mask=iota < m)
            return off_ + m
        off = lax.fori_loop(0, 64, pad_body, off)

        for g in range(3):
            dst = jnp.where(g * 1024 < off, wbase + fwords + g * 1024,
                            wbase + trash_off)
            pltpu.sync_copy(
                stage.at[pl.ds(g * 1024, 1024)],
                list_hbm.at[pl.ds(pl.multiple_of(dst, _CHUNK), 1024)])
        fwords = fwords + off

        nb = jnp.full((16,), fwords // 1024, jnp.int32)

        def cnt_body(i, carry_):
            stage[pl.ds(i * 16, 16)] = nb
            return carry_
        lax.fori_loop(0, 8, cnt_body, 0)
        pltpu.sync_copy(
            stage.at[pl.ds(0, _CHUNK)],
            list_hbm.at[pl.ds(pl.multiple_of(wbase + cnt_off, _CHUNK),
                              _CHUNK)])

    return filter_kernel, cap_chunks


def _make_acc_kernel(npad, hid, cap_chunks):
    rpt = npad // _NW
    cap_words = cap_chunks * _CHUNK
    cnt_off = (cap_chunks - 1) * _CHUNK
    mesh = plsc.VectorSubcoreMesh(core_axis_name="c", subcore_axis_name="s")

    @functools.partial(
        pl.kernel, mesh=mesh,
        compiler_params=_SC_PARAMS,
        out_type=jax.ShapeDtypeStruct((npad, hid), jnp.float32),
        scratch_types=[
            pltpu.VMEM((_CHUNK,), jnp.int32),
            pltpu.VMEM((1024,), jnp.int32),
            pltpu.VMEM((1024,), jnp.int32),
            pltpu.VMEM((npad // _NW, hid), jnp.float32),
            pltpu.VMEM((_CHUNK, hid), jnp.float32),
            pltpu.VMEM((_CHUNK, hid), jnp.float32),
            pltpu.SemaphoreType.DMA,
            pltpu.SemaphoreType.DMA,
        ],
    )
    def acc_kernel(h_hbm, list_hbm, out_hbm, cstage, lstage, gbuf, acc,
                   rows_a, rows_b, sem0, sem1):
        c = lax.axis_index("c")
        s = lax.axis_index("s")
        w = c * _NS + s
        wbase = w * cap_words

        pltpu.sync_copy(
            list_hbm.at[pl.ds(pl.multiple_of(wbase + cnt_off, _CHUNK),
                              _CHUNK)], cstage)
        nblocks = cstage[pl.ds(0, 16)][0]

        def zero_body(i, carry_):
            acc[i // 8, pl.ds((i % 8) * 16, 16)] = jnp.zeros((16,),
                                                             jnp.float32)
            return carry_
        lax.fori_loop(0, rpt * (hid // 16), zero_body, 0)

        rows = (rows_a, rows_b)
        sems = (sem0, sem1)
        nq = hid // 16

        def block_body(b, carry):
            pltpu.sync_copy(
                list_hbm.at[pl.ds(pl.multiple_of(wbase + b * 1024, _CHUNK),
                                  1024)], lstage)

            def unpack_body(k, carry_):
                gbuf[pl.ds(k * 16, 16)] = lstage[pl.ds(k * 16, 16)] // 512
                return carry_
            lax.fori_loop(0, 64, unpack_body, 0)

            pltpu.async_copy(h_hbm.at[gbuf.at[pl.ds(0, _CHUNK)]], rows[0],
                             sems[0])

            def pair_body(jj, carry_):
                for par in range(2):
                    j = jj * 2 + par
                    cur, nxt = par, 1 - par
                    jn = jnp.minimum(j + 1, 7)
                    pltpu.async_copy(
                        h_hbm.at[gbuf.at[pl.ds(jn * _CHUNK, _CHUNK)]],
                        rows[nxt], sems[nxt])
                    pltpu.make_async_copy(
                        h_hbm.at[pl.ds(0, _CHUNK)], rows[cur],
                        sems[cur]).wait()
                    rj = rows[cur]

                    @plsc.parallel_loop(0, 8, step=1, unroll=2)
                    def grp_body(g):
                        dv = lstage[pl.ds(j * 128 + g * 16, 16)] % 512
                        rbase = g * 16
                        for lane in range(16):
                            dst = dv[lane]
                            for cb in range(nq):
                                data = rj[rbase + lane, pl.ds(cb * 16, 16)]
                                plsc.addupdate(
                                    acc.at[dst, pl.ds(cb * 16, 16)], data)
                return carry_
            lax.fori_loop(0, 4, pair_body, 0)
            pltpu.make_async_copy(
                h_hbm.at[pl.ds(0, _CHUNK)], rows[0], sems[0]).wait()
            return carry
        lax.fori_loop(0, nblocks, block_body, 0)

        pltpu.sync_copy(acc, out_hbm.at[pl.ds(w * rpt, rpt)])

    return acc_kernel


def _make_bn(n, npad, hid):
    def _bn_body(p_ref, g_ref, b_ref, z_ref):
        o = p_ref[:n, :]
        mean = jnp.mean(o, axis=0, keepdims=True)
        d = o - mean
        var = jnp.mean(d * d, axis=0, keepdims=True)
        zn = d * lax.rsqrt(var + _EPS) * g_ref[...] + b_ref[...]
        z_ref[:n, :] = jnp.maximum(zn, 0.0)
        z_ref[n:, :] = jnp.zeros((npad - n, hid), jnp.float32)

    return pl.pallas_call(
        _bn_body,
        out_shape=jax.ShapeDtypeStruct((npad, hid), jnp.float32),
    )


def _make_seg(n, dim):
    def _seg_body(b_ref, z_ref, g_ref):
        ids = jnp.broadcast_to(b_ref[...], (_NUM_GRAPHS, n))
        onehot = (ids == lax.broadcasted_iota(jnp.int32, (_NUM_GRAPHS, n), 0)
                  ).astype(jnp.float32)
        g_ref[...] = lax.dot_general(
            onehot, z_ref[...], (((1,), (0,)), ((), ())),
            preferred_element_type=jnp.float32)

    return pl.pallas_call(
        _seg_body,
        out_shape=jax.ShapeDtypeStruct((_NUM_GRAPHS, dim), jnp.float32),
    )


def kernel(x, edge_index, edge_type, batch, W0, W1, W2, g0, b0, g1, b1, g2,
           b2):
    n, in_dim = x.shape
    ne = edge_index.shape[1]
    num_rel, _, hid = W0.shape
    id_dim = W1.shape[1] - hid

    npad = ((n + 2047) // 2048) * 2048
    ne_pad = ((ne + _BLK_EDGES - 1) // _BLK_EDGES) * _BLK_EDGES

    src = edge_index[0].astype(jnp.int32)
    dst = edge_index[1].astype(jnp.int32)
    et = edge_type.astype(jnp.int32)
    pad = ne_pad - ne
    gidx = jnp.concatenate(
        [et * npad + dst,
         jnp.full((pad,), npad - 1, jnp.int32)]).reshape(-1, _CHUNK)
    srcp = jnp.concatenate(
        [src, jnp.full((pad,), npad - 1, jnp.int32)]).reshape(-1, _CHUNK)

    x_pad = jnp.pad(x, ((0, npad - n), (0, 0)))
    filter_call, cap_chunks = _make_filter_kernel(npad, ne_pad)
    elist = filter_call(gidx, srcp)
    acc_call = _make_acc_kernel(npad, hid, cap_chunks)
    bn_call = _make_bn(n, npad, hid)

    Ws = (W0, W1, W2)
    gs = (g0, g1, g2)
    bs = (b0, b1, b2)
    zin = x_pad
    zs = []
    for l in range(3):
        H = _relu_mm(zin, Ws[l], bm=1024)
        out = acc_call(H.reshape(num_rel * npad, hid), elist)
        z = bn_call(out, gs[l].reshape(1, hid), bs[l].reshape(1, hid))
        zs.append(z[:n])
        if l < 2:
            zin = jnp.concatenate([x_pad[:, :id_dim], z], axis=1)

    z_cat = jnp.concatenate(zs, axis=1)
    g_cat = _make_seg(n, 3 * hid)(
        batch.reshape(1, n).astype(jnp.int32), z_cat)
    return (z_cat, g_cat)


# speedup vs baseline: 2.0335x; 2.0335x over previous
import functools

import jax
import jax.numpy as jnp
from jax import lax
from jax.experimental import pallas as pl
from jax.experimental.pallas import tpu as pltpu
from jax.experimental.pallas import tpu_sc as plsc

_NUM_GRAPHS = 16
_EPS = 1e-5
_NC = 2
_NS = 16
_CHUNK = 128


def _mm_body(z_ref, w_ref, h_ref):
    h_ref[0] = jnp.maximum(
        lax.dot_general(z_ref[...], w_ref[0], (((1,), (0,)), ((), ())),
                        preferred_element_type=jnp.float32),
        0.0)


def _relu_mm(zin, W, bm):
    npad, d = zin.shape
    num_rel, _, hid = W.shape
    return pl.pallas_call(
        _mm_body,
        grid=(npad // bm, num_rel),
        in_specs=[
            pl.BlockSpec((bm, d), lambda i, r: (i, 0)),
            pl.BlockSpec((1, d, hid), lambda i, r: (r, 0, 0)),
        ],
        out_specs=pl.BlockSpec((1, bm, hid), lambda i, r: (r, i, 0)),
        out_shape=jax.ShapeDtypeStruct((num_rel, npad, hid), jnp.float32),
    )(zin, W)


def _make_edge_kernel(npad, hid, chunks_per_worker, nrows_tab):
    rows_per_tile = npad // _NS

    mesh = plsc.VectorSubcoreMesh(core_axis_name="c", subcore_axis_name="s")

    half = chunks_per_worker // 2

    @functools.partial(
        pl.kernel, mesh=mesh,
        out_type=jax.ShapeDtypeStruct((_NC, npad, hid), jnp.float32),
        scratch_types=[
            pltpu.VMEM((half, _CHUNK), jnp.int32),
            pltpu.VMEM((half, _CHUNK), jnp.int32),
            pltpu.VMEM((_CHUNK, hid), jnp.float32),
            pltpu.VMEM((_CHUNK, hid), jnp.float32),
            pltpu.VMEM_SHARED((npad, hid), jnp.float32),
            pltpu.SemaphoreType.DMA,
            pltpu.SemaphoreType.DMA,
        ],
    )
    def edge_kernel(h_hbm, gidx_hbm, src_hbm, out_hbm, gidx_v, src_v, rows_a,
                    rows_b, acc_sh, sem0, sem1):
        c = lax.axis_index("c")
        s = lax.axis_index("s")
        wid = c * _NS + s
        row_base = wid * chunks_per_worker

        def _zero_body(i, carry):
            r = i // (hid // 16)
            col = (i % (hid // 16)) * 16
            rows_a[r, pl.ds(col, 16)] = jnp.zeros((16,), jnp.float32)
            return carry
        lax.fori_loop(0, _CHUNK * (hid // 16), _zero_body, 0)
        for t in range(rows_per_tile // _CHUNK):
            pltpu.sync_copy(
                rows_a,
                acc_sh.at[pl.ds(s * rows_per_tile + t * _CHUNK, _CHUNK)])
        plsc.subcore_barrier()

        for h in range(2):
            pltpu.sync_copy(
                gidx_hbm.at[pl.ds(row_base + h * half, half)], gidx_v)
            pltpu.sync_copy(
                src_hbm.at[pl.ds(row_base + h * half, half)], src_v)
            pltpu.async_copy(h_hbm.at[gidx_v.at[0]], rows_a, sem0)

            def _body(i, carry):
                k0 = 2 * i
                pltpu.async_copy(h_hbm.at[gidx_v.at[k0 + 1]], rows_b, sem1)
                pltpu.make_async_copy(
                    h_hbm.at[pl.ds(0, _CHUNK)], rows_a, sem0).wait()
                pltpu.sync_copy(rows_a, acc_sh.at[src_v.at[k0]], add=True)
                k2 = jnp.minimum(k0 + 2, half - 2)
                pltpu.async_copy(h_hbm.at[gidx_v.at[k2]], rows_a, sem0)
                pltpu.make_async_copy(
                    h_hbm.at[pl.ds(0, _CHUNK)], rows_b, sem1).wait()
                pltpu.sync_copy(rows_b, acc_sh.at[src_v.at[k0 + 1]],
                                add=True)
                return carry
            lax.fori_loop(0, half // 2, _body, 0)
            pltpu.make_async_copy(
                h_hbm.at[pl.ds(0, _CHUNK)], rows_a, sem0).wait()
        plsc.subcore_barrier()

        for t in range(rows_per_tile // _CHUNK):
            off = s * rows_per_tile + t * _CHUNK
            pltpu.sync_copy(acc_sh.at[pl.ds(off, _CHUNK)], rows_a)
            pltpu.sync_copy(rows_a, out_hbm.at[c, pl.ds(off, _CHUNK)])

    return edge_kernel


def _make_bn(n, npad, hid):
    def _bn_body(p_ref, g_ref, b_ref, z_ref):
        o = p_ref[0, :n, :] + p_ref[1, :n, :]
        mean = jnp.mean(o, axis=0, keepdims=True)
        d = o - mean
        var = jnp.mean(d * d, axis=0, keepdims=True)
        zn = d * lax.rsqrt(var + _EPS) * g_ref[...] + b_ref[...]
        z_ref[:n, :] = jnp.maximum(zn, 0.0)
        z_ref[n:, :] = jnp.zeros((npad - n, hid), jnp.float32)

    return pl.pallas_call(
        _bn_body,
        out_shape=jax.ShapeDtypeStruct((npad, hid), jnp.float32),
    )


def _make_seg(n, dim):
    def _seg_body(b_ref, z_ref, g_ref):
        ids = jnp.broadcast_to(b_ref[...], (_NUM_GRAPHS, n))
        onehot = (ids == lax.broadcasted_iota(jnp.int32, (_NUM_GRAPHS, n), 0)
                  ).astype(jnp.float32)
        g_ref[...] = lax.dot_general(
            onehot, z_ref[...], (((1,), (0,)), ((), ())),
            preferred_element_type=jnp.float32)

    return pl.pallas_call(
        _seg_body,
        out_shape=jax.ShapeDtypeStruct((_NUM_GRAPHS, dim), jnp.float32),
    )


def kernel(x, edge_index, edge_type, batch, W0, W1, W2, g0, b0, g1, b1, g2,
           b2):
    n, in_dim = x.shape
    ne = edge_index.shape[1]
    num_rel, _, hid = W0.shape
    id_dim = W1.shape[1] - hid

    nw = _NC * _NS
    npad = ((n + 2047) // 2048) * 2048
    epw = ((ne + nw * _CHUNK * 8 - 1) // (nw * _CHUNK * 8)) * _CHUNK * 8
    ne_pad = epw * nw
    chunks_per_worker = epw // _CHUNK

    src = edge_index[0].astype(jnp.int32)
    dst = edge_index[1].astype(jnp.int32)
    et = edge_type.astype(jnp.int32)
    pad = ne_pad - ne
    gidx = jnp.concatenate(
        [et * npad + dst, jnp.zeros((pad,), jnp.int32)]).reshape(-1, _CHUNK)
    srcp = jnp.concatenate(
        [src, jnp.full((pad,), npad - 1, jnp.int32)]).reshape(-1, _CHUNK)

    x_pad = jnp.pad(x, ((0, npad - n), (0, 0)))
    edge_call = _make_edge_kernel(npad, hid, chunks_per_worker,
                                  num_rel * npad)
    bn_call = _make_bn(n, npad, hid)

    Ws = (W0, W1, W2)
    gs = (g0, g1, g2)
    bs = (b0, b1, b2)
    zin = x_pad
    zs = []
    for l in range(3):
        H = _relu_mm(zin, Ws[l], bm=1024)
        parts = edge_call(H.reshape(num_rel * npad, hid), gidx, srcp)
        z = bn_call(parts, gs[l].reshape(1, hid), bs[l].reshape(1, hid))
        zs.append(z[:n])
        if l < 2:
            zin = jnp.concatenate([x_pad[:, :id_dim], z], axis=1)

    z_cat = jnp.concatenate(zs, axis=1)
    g_cat = _make_seg(n, 3 * hid)(
        batch.reshape(1, n).astype(jnp.int32), z_cat)
    return (z_cat, g_cat)
